# baseline (speedup 1.0000x reference)
import jax
import jax.numpy as jnp
from jax import lax
from jax.experimental import pallas as pl
from jax.experimental.pallas import tpu as pltpu
from jax.experimental.pallas import tpu_sc as plsc

EMBED = 32
ROWS = 16384
COLS = 26

_info = plsc.get_sparse_core_info()
NC, NS, L = _info.num_cores, _info.num_subcores, _info.num_lanes
NW = NC * NS
ROWS_PER_W = ROWS // NW
R = 64
N_CHUNKS = ROWS_PER_W // R


def _body(table_hbm, x_hbm, out_hbm, idx0_v, idx1_v, rows0_v, rows1_v,
          gsem0, gsem1, osem0, osem1):
    wid = lax.axis_index("s") * NC + lax.axis_index("c")
    base = wid * ROWS_PER_W
    idx = (idx0_v, idx1_v)
    rows = (rows0_v, rows1_v)
    gsem = (gsem0, gsem1)
    osem = (osem0, osem1)

    def stage(g, b):
        cb = base + g * R
        pltpu.sync_copy(x_hbm.at[pl.ds(cb, R)], idx[b])

        def fire(r, _):
            pltpu.async_copy(table_hbm.at[idx[b].at[r]], rows[b].at[r], gsem[b])
            return 0

        lax.fori_loop(0, R, fire, 0)

    def finish(g, b):
        cb = base + g * R

        def drain(r, _):
            pltpu.make_async_copy(
                table_hbm.at[idx[b].at[r]], rows[b].at[r], gsem[b]).wait()
            return 0

        lax.fori_loop(0, R, drain, 0)

        def fix(r, _):
            va = idx[b][r, pl.ds(0, L)]
            vb = idx[b][r, pl.ds(COLS - L, L)]
            ma = jnp.where(va == 0, 0.0, 1.0)
            mb = jnp.where(vb == 0, 0.0, 1.0)
            for j in range(COLS):
                if j < L:
                    m = ma[jnp.full((L,), j, jnp.int32)]
                else:
                    m = mb[jnp.full((L,), j - (COLS - L), jnp.int32)]
                rows[b][r, j, pl.ds(0, L)] = rows[b][r, j, pl.ds(0, L)] * m
                rows[b][r, j, pl.ds(L, L)] = rows[b][r, j, pl.ds(L, L)] * m
            return 0

        lax.fori_loop(0, R, fix, 0)
        pltpu.async_copy(rows[b], out_hbm.at[pl.ds(cb, R)], osem[b])

    def wait_out(g, b):
        cb = base + g * R
        pltpu.make_async_copy(rows[b], out_hbm.at[pl.ds(cb, R)], osem[b]).wait()

    stage(0, 0)
    for g in range(N_CHUNKS):
        b = g % 2
        if g + 1 < N_CHUNKS:
            b2 = (g + 1) % 2
            if g >= 1:
                wait_out(g - 1, b2)
            stage(g + 1, b2)
        finish(g, b)
    if N_CHUNKS >= 2:
        wait_out(N_CHUNKS - 2, (N_CHUNKS - 2) % 2)
    wait_out(N_CHUNKS - 1, (N_CHUNKS - 1) % 2)


@jax.jit
def _gather(table, x):
    mesh = plsc.VectorSubcoreMesh(core_axis_name="c", subcore_axis_name="s")
    return pl.kernel(
        _body,
        out_type=jax.ShapeDtypeStruct((ROWS, COLS, EMBED), jnp.float32),
        mesh=mesh,
        scratch_types=[
            pltpu.VMEM((R, COLS), jnp.int32),
            pltpu.VMEM((R, COLS), jnp.int32),
            pltpu.VMEM((R, COLS, EMBED), jnp.float32),
            pltpu.VMEM((R, COLS, EMBED), jnp.float32),
            pltpu.SemaphoreType.DMA,
            pltpu.SemaphoreType.DMA,
            pltpu.SemaphoreType.DMA,
            pltpu.SemaphoreType.DMA,
        ],
        compiler_params=pltpu.CompilerParams(use_tc_tiling_on_sc=False),
    )(table, x)


def kernel(x, table):
    return _gather(table, x.astype(jnp.int32))


# speedup vs baseline: 1.0136x; 1.0136x over previous
import jax
import jax.numpy as jnp
from jax import lax
from jax.experimental import pallas as pl
from jax.experimental.pallas import tpu as pltpu
from jax.experimental.pallas import tpu_sc as plsc

EMBED = 32
ROWS = 16384
COLS = 26
XROWS = 4096
XCOLS = 104

_info = plsc.get_sparse_core_info()
NC, NS, L = _info.num_cores, _info.num_subcores, _info.num_lanes
NW = NC * NS
ROWS_PER_W = XROWS // NW
R = 16
N_CHUNKS = ROWS_PER_W // R


def _body(table_hbm, x_hbm, out_hbm, idx0_v, idx1_v, rows0_v, rows1_v,
          gsem0, gsem1, osem0, osem1):
    wid = lax.axis_index("s") * NC + lax.axis_index("c")
    base = wid * ROWS_PER_W
    idx = (idx0_v, idx1_v)
    rows = (rows0_v, rows1_v)
    gsem = (gsem0, gsem1)
    osem = (osem0, osem1)

    def stage(g, b):
        cb = base + g * R
        pltpu.sync_copy(x_hbm.at[pl.ds(cb, R)], idx[b])

        def fire(r, _):
            pltpu.async_copy(table_hbm.at[idx[b].at[r]], rows[b].at[r], gsem[b])
            return 0

        lax.fori_loop(0, R, fire, 0)

    def finish(g, b):
        cb = base + g * R

        def drain(r, _):
            pltpu.make_async_copy(
                table_hbm.at[idx[b].at[r]], rows[b].at[r], gsem[b]).wait()
            return 0

        lax.fori_loop(0, R, drain, 0)

        def fix(r, _):
            m = []
            for k in range(6):
                v = idx[b][r, pl.ds(k * L, L)]
                m.append(jnp.where(v == 0, 0.0, 1.0))
            v = idx[b][r, pl.ds(XCOLS - L, L)]
            m.append(jnp.where(v == 0, 0.0, 1.0))
            for j in range(XCOLS):
                if j < 6 * L:
                    mj = m[j // L][jnp.full((L,), j % L, jnp.int32)]
                else:
                    mj = m[6][jnp.full((L,), j - (XCOLS - L), jnp.int32)]
                rows[b][r, j, pl.ds(0, L)] = rows[b][r, j, pl.ds(0, L)] * mj
                rows[b][r, j, pl.ds(L, L)] = rows[b][r, j, pl.ds(L, L)] * mj
            return 0

        lax.fori_loop(0, R, fix, 0)
        pltpu.async_copy(rows[b], out_hbm.at[pl.ds(cb, R)], osem[b])

    def wait_out(g, b):
        cb = base + g * R
        pltpu.make_async_copy(rows[b], out_hbm.at[pl.ds(cb, R)], osem[b]).wait()

    stage(0, 0)
    for g in range(N_CHUNKS):
        b = g % 2
        if g + 1 < N_CHUNKS:
            b2 = (g + 1) % 2
            if g >= 1:
                wait_out(g - 1, b2)
            stage(g + 1, b2)
        finish(g, b)
    if N_CHUNKS >= 2:
        wait_out(N_CHUNKS - 2, (N_CHUNKS - 2) % 2)
    wait_out(N_CHUNKS - 1, (N_CHUNKS - 1) % 2)


@jax.jit
def _gather(table, x):
    mesh = plsc.VectorSubcoreMesh(core_axis_name="c", subcore_axis_name="s")
    return pl.kernel(
        _body,
        out_type=jax.ShapeDtypeStruct((XROWS, XCOLS, EMBED), jnp.float32),
        mesh=mesh,
        scratch_types=[
            pltpu.VMEM((R, XCOLS), jnp.int32),
            pltpu.VMEM((R, XCOLS), jnp.int32),
            pltpu.VMEM((R, XCOLS, EMBED), jnp.float32),
            pltpu.VMEM((R, XCOLS, EMBED), jnp.float32),
            pltpu.SemaphoreType.DMA,
            pltpu.SemaphoreType.DMA,
            pltpu.SemaphoreType.DMA,
            pltpu.SemaphoreType.DMA,
        ],
        compiler_params=pltpu.CompilerParams(use_tc_tiling_on_sc=False),
    )(table, x)


def kernel(x, table):
    xr = x.astype(jnp.int32).reshape(XROWS, XCOLS)
    return _gather(table, xr).reshape(ROWS, COLS, EMBED)
